# Initial kernel scaffold; baseline (speedup 1.0000x reference)
import jax
import jax.numpy as jnp
from jax.experimental import pallas as pl


def kernel(Z, W_m2v, W_mag, W_oli, P_m2v_w, P_m2v_b, P_mag_w, P_mag_b, P_oli_w, P_oli_b):
    raise NotImplementedError("write your pallas kernel here")


# speedup vs baseline: 3.8356x; 3.8356x over previous
import functools

import jax
import jax.numpy as jnp
from jax import lax
from jax.experimental import pallas as pl
from jax.experimental.pallas import tpu as pltpu
from jax.experimental.pallas import tpu_sc as plsc

VOCAB = 119
D_OUT = 64
NC = 2
NS = 16
NW = NC * NS
CHUNK = 128


def _project_tables(W1, P1t, b1, W2, P2t, b2, W3, P3t, b3):
    def body(w1, p1, c1, w2, p2, c2, w3, p3, c3, t1, t2, t3):
        dn = (((1,), (0,)), ((), ()))
        for w, p, c, t in ((w1, p1, c1, t1), (w2, p2, c2, t2), (w3, p3, c3, t3)):
            t[...] = lax.dot_general(w[...], p[...], dn,
                                     preferred_element_type=jnp.float32,
                                     precision=lax.Precision.HIGHEST) + c[...]

    out_shape = [jax.ShapeDtypeStruct((VOCAB, D_OUT), jnp.float32)] * 3
    return pl.pallas_call(body, out_shape=out_shape)(
        W1, P1t, b1, W2, P2t, b2, W3, P3t, b3)


@functools.partial(jax.jit, static_argnames=("b_tot",))
def _sc_gather3(idx, T1, T2, T3, b_tot):
    b_per_w = b_tot // NW
    n_chunks = b_per_w // CHUNK
    mesh = plsc.VectorSubcoreMesh(core_axis_name="c", subcore_axis_name="s")

    @functools.partial(
        pl.kernel,
        out_type=(jax.ShapeDtypeStruct((b_tot, D_OUT), jnp.float32),) * 3,
        mesh=mesh,
        compiler_params=pltpu.CompilerParams(use_tc_tiling_on_sc=False),
        scratch_types=[
            pltpu.VMEM((CHUNK,), jnp.int32),
            pltpu.VMEM((CHUNK, D_OUT), jnp.float32),
            pltpu.VMEM((CHUNK, D_OUT), jnp.float32),
            pltpu.VMEM((CHUNK, D_OUT), jnp.float32),
            pltpu.SemaphoreType.DMA,
        ],
    )
    def k(idx_hbm, t1_hbm, t2_hbm, t3_hbm, o1, o2, o3, idx_v, r1, r2, r3, sem):
        wid = lax.axis_index("s") * NC + lax.axis_index("c")
        base = wid * b_per_w

        def step(i, _):
            off = base + i * CHUNK
            pltpu.sync_copy(idx_hbm.at[pl.ds(off, CHUNK)], idx_v)
            g1 = pltpu.async_copy(t1_hbm.at[idx_v], r1, sem)
            g2 = pltpu.async_copy(t2_hbm.at[idx_v], r2, sem)
            g3 = pltpu.async_copy(t3_hbm.at[idx_v], r3, sem)
            g1.wait()
            g2.wait()
            g3.wait()
            pltpu.sync_copy(r1, o1.at[pl.ds(off, CHUNK)])
            pltpu.sync_copy(r2, o2.at[pl.ds(off, CHUNK)])
            pltpu.sync_copy(r3, o3.at[pl.ds(off, CHUNK)])
            return 0

        lax.fori_loop(0, n_chunks, step, 0)

    return k(idx, T1, T2, T3)


def kernel(Z, W_m2v, W_mag, W_oli, P_m2v_w, P_m2v_b, P_mag_w, P_mag_b,
           P_oli_w, P_oli_b):
    B, S = Z.shape
    T1, T2, T3 = _project_tables(
        W_m2v, P_m2v_w.T, P_m2v_b.reshape(1, D_OUT),
        W_mag, P_mag_w.T, P_mag_b.reshape(1, D_OUT),
        W_oli, P_oli_w.T, P_oli_b.reshape(1, D_OUT))
    idx = Z.reshape(-1).astype(jnp.int32)
    v1, v2, v3 = _sc_gather3(idx, T1, T2, T3, b_tot=B * S)
    return (v1.reshape(B, S, D_OUT), v2.reshape(B, S, D_OUT),
            v3.reshape(B, S, D_OUT))
